# baseline (speedup 1.0000x reference)
import functools

import jax
import jax.numpy as jnp
from jax import lax
from jax.experimental import pallas as pl
from jax.experimental.pallas import tpu as pltpu
from jax.experimental.pallas import tpu_sc as plsc

DIM = 64
NC = 2
NS = 16
NW = NC * NS
BLK = 128


def _make_gather(batch, hist):
    assert batch == NW * BLK and DIM == 64
    mesh = plsc.VectorSubcoreMesh(core_axis_name="c", subcore_axis_name="s")

    scratch = [
        pltpu.VMEM((hist, BLK), jnp.int32),
        pltpu.VMEM((BLK, DIM), jnp.float32),
        pltpu.VMEM((DIM // 8, 8 * BLK), jnp.float32),
        pltpu.SemaphoreType.DMA,
    ]

    @functools.partial(
        pl.kernel,
        out_type=jax.ShapeDtypeStruct((hist, DIM // 8, NW, 8 * BLK),
                                      jnp.float32),
        mesh=mesh,
        scratch_types=scratch,
        compiler_params=pltpu.CompilerParams(
            use_tc_tiling_on_sc=False, needs_layout_passes=False),
    )
    def gather_kernel(idx_hbm, table_hbm, out_hbm, idx_v, rows_v, outt_v, sem):
        wid = lax.axis_index("s") * NC + lax.axis_index("c")

        pltpu.sync_copy(idx_hbm.at[:, pl.ds(wid * BLK, BLK)], idx_v)

        iota = lax.broadcasted_iota(jnp.int32, (16,), 0)
        lanes = [iota + b16 * 16 for b16 in range(BLK // 16)]

        @pl.loop(0, hist)
        def _(h):
            pltpu.async_copy(table_hbm.at[idx_v.at[h]], rows_v, sem).wait()
            @pl.loop(0, DIM // 8)
            def _(tr):
                @pl.loop(0, 8)
                def _(es):
                    ev = jnp.full((16,), tr * 8 + es, jnp.int32)
                    for b16 in range(BLK // 16):
                        rv = plsc.load_gather(rows_v, [lanes[b16], ev])
                        outt_v[tr, pl.ds(es * BLK + b16 * 16, 16)] = rv
            pltpu.sync_copy(outt_v, out_hbm.at[h, :, wid])

    return gather_kernel


@jax.jit
def kernel(x, action_emb_weight):
    b, h = x.shape
    xt = jnp.transpose(x, (1, 0)).astype(jnp.int32)
    out5 = _make_gather(b, h)(xt, action_emb_weight)
    out = out5.reshape(h, DIM // 8, NW, 8, BLK)
    return jnp.transpose(out, (2, 4, 0, 1, 3)).reshape(b, h, DIM)


# speedup vs baseline: 1.0391x; 1.0391x over previous
import functools

import jax
import jax.numpy as jnp
from jax import lax
from jax.experimental import pallas as pl
from jax.experimental.pallas import tpu as pltpu
from jax.experimental.pallas import tpu_sc as plsc

DIM = 64
NC = 2
NS = 16
NW = NC * NS
BLK = 128


def _make_gather(batch, hist):
    assert batch == NW * BLK and DIM == 64
    mesh = plsc.VectorSubcoreMesh(core_axis_name="c", subcore_axis_name="s")

    scratch = [
        pltpu.VMEM((hist, BLK), jnp.int32),
        pltpu.VMEM((BLK, DIM), jnp.float32),
        pltpu.VMEM((BLK, DIM), jnp.float32),
        pltpu.VMEM((DIM // 8, 8 * BLK), jnp.float32),
        pltpu.VMEM((DIM // 8, 8 * BLK), jnp.float32),
        pltpu.SemaphoreType.DMA,
        pltpu.SemaphoreType.DMA,
        pltpu.SemaphoreType.DMA,
        pltpu.SemaphoreType.DMA,
    ]

    @functools.partial(
        pl.kernel,
        out_type=jax.ShapeDtypeStruct((hist, DIM // 8, NW, 8 * BLK),
                                      jnp.float32),
        mesh=mesh,
        scratch_types=scratch,
        compiler_params=pltpu.CompilerParams(
            use_tc_tiling_on_sc=False, needs_layout_passes=False),
    )
    def gather_kernel(idx_hbm, table_hbm, out_hbm, idx_v,
                      rows0, rows1, outt0, outt1, g0, g1, s0, s1):
        rows = (rows0, rows1)
        outt = (outt0, outt1)
        gsem = (g0, g1)
        ssem = (s0, s1)
        wid = lax.axis_index("s") * NC + lax.axis_index("c")

        pltpu.sync_copy(idx_hbm.at[:, pl.ds(wid * BLK, BLK)], idx_v)

        iota = lax.broadcasted_iota(jnp.int32, (16,), 0)
        lanes = [iota + b16 * 16 for b16 in range(BLK // 16)]

        def start_gather(h, p):
            pltpu.make_async_copy(
                table_hbm.at[idx_v.at[h]], rows[p], gsem[p]).start()

        def wait_gather(h, p):
            pltpu.make_async_copy(
                table_hbm.at[idx_v.at[h]], rows[p], gsem[p]).wait()

        def start_store(h, p):
            pltpu.make_async_copy(
                outt[p], out_hbm.at[h, :, wid], ssem[p]).start()

        def wait_store(h, p):
            pltpu.make_async_copy(
                outt[p], out_hbm.at[h, :, wid], ssem[p]).wait()

        def transpose(p):
            for tr in range(DIM // 8):
                for es in range(8):
                    ev = jnp.full((16,), tr * 8 + es, jnp.int32)
                    for b16 in range(BLK // 16):
                        rv = plsc.load_gather(rows[p], [lanes[b16], ev])
                        outt[p][tr, pl.ds(es * BLK + b16 * 16, 16)] = rv

        start_gather(0, 0)
        start_gather(1, 1)

        @pl.loop(0, hist // 2)
        def _(g):
            for par in (0, 1):
                h = 2 * g + par

                wait_gather(h, par)

                @pl.when(g > 0)
                def _():
                    wait_store(h - 2, par)

                transpose(par)
                start_store(h, par)

                @pl.when(h + 2 < hist)
                def _():
                    start_gather(h + 2, par)

        wait_store(hist - 2, 0)
        wait_store(hist - 1, 1)

    return gather_kernel


@jax.jit
def kernel(x, action_emb_weight):
    b, h = x.shape
    xt = jnp.transpose(x, (1, 0)).astype(jnp.int32)
    out5 = _make_gather(b, h)(xt, action_emb_weight)
    out = out5.reshape(h, DIM // 8, NW, 8, BLK)
    return jnp.transpose(out, (2, 4, 0, 1, 3)).reshape(b, h, DIM)


# speedup vs baseline: 2.0881x; 2.0095x over previous
import functools

import jax
import jax.numpy as jnp
from jax import lax
from jax.experimental import pallas as pl
from jax.experimental.pallas import tpu as pltpu
from jax.experimental.pallas import tpu_sc as plsc

DIM = 64
NC = 2
NS = 16
NW = NC * NS
BLK = 128


def _make_gather(batch, hist):
    assert batch == NW * BLK and DIM == 64
    mesh = plsc.VectorSubcoreMesh(core_axis_name="c", subcore_axis_name="s")

    scratch = [
        pltpu.VMEM((hist, BLK), jnp.int32),
        pltpu.VMEM((BLK, DIM), jnp.float32),
        pltpu.VMEM((BLK, DIM), jnp.float32),
        pltpu.VMEM((DIM // 8, 8 * BLK), jnp.float32),
        pltpu.VMEM((DIM // 8, 8 * BLK), jnp.float32),
        pltpu.SemaphoreType.DMA,
        pltpu.SemaphoreType.DMA,
        pltpu.SemaphoreType.DMA,
        pltpu.SemaphoreType.DMA,
    ]

    @functools.partial(
        pl.kernel,
        out_type=jax.ShapeDtypeStruct((hist, DIM // 8, NW, 8 * BLK),
                                      jnp.float32),
        mesh=mesh,
        scratch_types=scratch,
        compiler_params=pltpu.CompilerParams(
            use_tc_tiling_on_sc=False, needs_layout_passes=False),
    )
    def gather_kernel(idx_hbm, table_hbm, out_hbm, idx_v,
                      rows0, rows1, outt0, outt1, g0, g1, s0, s1):
        rows = (rows0, rows1)
        outt = (outt0, outt1)
        gsem = (g0, g1)
        ssem = (s0, s1)
        wid = lax.axis_index("s") * NC + lax.axis_index("c")

        pltpu.sync_copy(idx_hbm.at[:, pl.ds(wid * BLK, BLK)], idx_v)

        iota = lax.broadcasted_iota(jnp.int32, (16,), 0)
        rowv = [iota + b16 * 16 for b16 in range(BLK // 16)]

        def start_gather(h, p):
            pltpu.make_async_copy(
                table_hbm.at[idx_v.at[h]], rows[p], gsem[p]).start()

        def wait_gather(h, p):
            pltpu.make_async_copy(
                table_hbm.at[idx_v.at[h]], rows[p], gsem[p]).wait()

        def start_store(h, p):
            pltpu.make_async_copy(
                outt[p], out_hbm.at[h, :, wid], ssem[p]).start()

        def wait_store(h, p):
            pltpu.make_async_copy(
                outt[p], out_hbm.at[h, :, wid], ssem[p]).wait()

        def transpose(p):
            @pl.loop(0, 16)
            def _(k):
                m = jnp.bitwise_and(iota + k, 15)
                colv = [m + e16 * 16 for e16 in range(DIM // 16)]
                trv = [jnp.right_shift(m, 3) + e16 * 2
                       for e16 in range(DIM // 16)]
                cbase = jnp.left_shift(jnp.bitwise_and(m, 7), 7) + iota
                for b16 in range(BLK // 16):
                    cb = cbase + b16 * 16
                    for e16 in range(DIM // 16):
                        rv = plsc.load_gather(rows[p], [rowv[b16], colv[e16]])
                        plsc.store_scatter(outt[p], [trv[e16], cb], rv)

        start_gather(0, 0)
        start_gather(1, 1)

        @pl.loop(0, hist // 2)
        def _(g):
            for par in (0, 1):
                h = 2 * g + par

                wait_gather(h, par)

                @pl.when(g > 0)
                def _():
                    wait_store(h - 2, par)

                transpose(par)
                start_store(h, par)

                @pl.when(h + 2 < hist)
                def _():
                    start_gather(h + 2, par)

        wait_store(hist - 2, 0)
        wait_store(hist - 1, 1)

    return gather_kernel


@jax.jit
def kernel(x, action_emb_weight):
    b, h = x.shape
    xt = jnp.transpose(x, (1, 0)).astype(jnp.int32)
    out5 = _make_gather(b, h)(xt, action_emb_weight)
    out = out5.reshape(h, DIM // 8, NW, 8, BLK)
    return jnp.transpose(out, (2, 4, 0, 1, 3)).reshape(b, h, DIM)


# speedup vs baseline: 2.5829x; 1.2370x over previous
import functools

import jax
import jax.numpy as jnp
from jax import lax
from jax.experimental import pallas as pl
from jax.experimental.pallas import tpu as pltpu
from jax.experimental.pallas import tpu_sc as plsc

DIM = 64
NC = 2
NS = 16
NW = NC * NS
BLK = 128


def _make_gather(batch, hist):
    assert batch == NW * BLK and DIM == 64
    mesh = plsc.VectorSubcoreMesh(core_axis_name="c", subcore_axis_name="s")

    scratch = [
        pltpu.VMEM((hist, BLK), jnp.int32),
        pltpu.VMEM((BLK, DIM), jnp.float32),
        pltpu.VMEM((BLK, DIM), jnp.float32),
        pltpu.VMEM((DIM // 8, 8 * BLK), jnp.float32),
        pltpu.VMEM((DIM // 8, 8 * BLK), jnp.float32),
        pltpu.SemaphoreType.DMA,
        pltpu.SemaphoreType.DMA,
        pltpu.SemaphoreType.DMA,
        pltpu.SemaphoreType.DMA,
    ]

    @functools.partial(
        pl.kernel,
        out_type=jax.ShapeDtypeStruct((hist, DIM // 8, NW, 8 * BLK),
                                      jnp.float32),
        mesh=mesh,
        scratch_types=scratch,
        compiler_params=pltpu.CompilerParams(
            use_tc_tiling_on_sc=False, needs_layout_passes=False),
    )
    def gather_kernel(idx_hbm, table_hbm, out_hbm, idx_v,
                      rows0, rows1, outt0, outt1, g0, g1, s0, s1):
        rows = (rows0, rows1)
        outt = (outt0, outt1)
        gsem = (g0, g1)
        ssem = (s0, s1)
        wid = lax.axis_index("s") * NC + lax.axis_index("c")

        pltpu.sync_copy(idx_hbm.at[:, pl.ds(wid * BLK, BLK)], idx_v)

        iota = lax.broadcasted_iota(jnp.int32, (16,), 0)
        rowv = [iota + b16 * 16 for b16 in range(BLK // 16)]

        def start_gather(h, p):
            pltpu.make_async_copy(
                table_hbm.at[idx_v.at[h]], rows[p], gsem[p]).start()

        def wait_gather(h, p):
            pltpu.make_async_copy(
                table_hbm.at[idx_v.at[h]], rows[p], gsem[p]).wait()

        def start_store(h, p):
            pltpu.make_async_copy(
                outt[p], out_hbm.at[h, :, wid], ssem[p]).start()

        def wait_store(h, p):
            pltpu.make_async_copy(
                outt[p], out_hbm.at[h, :, wid], ssem[p]).wait()

        def transpose(p):
            @pl.loop(0, 16)
            def _(k):
                m = jnp.bitwise_and(iota + k, 15)
                colv = [m + e16 * 16 for e16 in range(DIM // 16)]
                trv = [jnp.right_shift(m, 3) + e16 * 2
                       for e16 in range(DIM // 16)]
                cbase = jnp.left_shift(jnp.bitwise_and(m, 7), 7) + iota
                prev = None
                for b16 in range(BLK // 16):
                    cur = (cbase + b16 * 16,
                           [plsc.load_gather(rows[p], [rowv[b16], colv[e16]])
                            for e16 in range(DIM // 16)])
                    if prev is not None:
                        pcb, prvs = prev
                        for e16 in range(DIM // 16):
                            plsc.store_scatter(outt[p], [trv[e16], pcb],
                                               prvs[e16])
                    prev = cur
                pcb, prvs = prev
                for e16 in range(DIM // 16):
                    plsc.store_scatter(outt[p], [trv[e16], pcb], prvs[e16])

        start_gather(0, 0)
        start_gather(1, 1)

        @pl.loop(0, hist // 2)
        def _(g):
            for par in (0, 1):
                h = 2 * g + par

                wait_gather(h, par)

                @pl.when(g > 0)
                def _():
                    wait_store(h - 2, par)

                transpose(par)
                start_store(h, par)

                @pl.when(h + 2 < hist)
                def _():
                    start_gather(h + 2, par)

        wait_store(hist - 2, 0)
        wait_store(hist - 1, 1)

    return gather_kernel


@jax.jit
def kernel(x, action_emb_weight):
    b, h = x.shape
    xt = jnp.transpose(x, (1, 0)).astype(jnp.int32)
    out5 = _make_gather(b, h)(xt, action_emb_weight)
    out = out5.reshape(h, DIM // 8, NW, 8, BLK)
    return jnp.transpose(out, (2, 4, 0, 1, 3)).reshape(b, h, DIM)


# speedup vs baseline: 2.6573x; 1.0288x over previous
import functools

import jax
import jax.numpy as jnp
from jax import lax
from jax.experimental import pallas as pl
from jax.experimental.pallas import tpu as pltpu
from jax.experimental.pallas import tpu_sc as plsc

DIM = 64
NC = 2
NS = 16
NW = NC * NS
BLK = 128


def _make_gather(batch, hist):
    assert batch == NW * BLK and DIM == 64
    mesh = plsc.VectorSubcoreMesh(core_axis_name="c", subcore_axis_name="s")

    scratch = [
        pltpu.VMEM((hist, BLK), jnp.int32),
        pltpu.VMEM((BLK, DIM), jnp.float32),
        pltpu.VMEM((BLK, DIM), jnp.float32),
        pltpu.VMEM((DIM // 8, 8 * BLK), jnp.float32),
        pltpu.VMEM((DIM // 8, 8 * BLK), jnp.float32),
        pltpu.SemaphoreType.DMA,
        pltpu.SemaphoreType.DMA,
        pltpu.SemaphoreType.DMA,
        pltpu.SemaphoreType.DMA,
    ]

    @functools.partial(
        pl.kernel,
        out_type=jax.ShapeDtypeStruct((hist, DIM // 8, NW, 8 * BLK),
                                      jnp.float32),
        mesh=mesh,
        scratch_types=scratch,
        compiler_params=pltpu.CompilerParams(
            use_tc_tiling_on_sc=False, needs_layout_passes=False),
    )
    def gather_kernel(idx_hbm, table_hbm, out_hbm, idx_v,
                      rows0, rows1, outt0, outt1, g0, g1, s0, s1):
        rows = (rows0, rows1)
        outt = (outt0, outt1)
        gsem = (g0, g1)
        ssem = (s0, s1)
        wid = lax.axis_index("s") * NC + lax.axis_index("c")

        pltpu.sync_copy(idx_hbm.at[:, pl.ds(wid * BLK, BLK)], idx_v)

        iota = lax.broadcasted_iota(jnp.int32, (16,), 0)
        rowv = [iota + b16 * 16 for b16 in range(BLK // 16)]

        def start_gather(h, p):
            pltpu.make_async_copy(
                table_hbm.at[idx_v.at[h]], rows[p], gsem[p]).start()

        def wait_gather(h, p):
            pltpu.make_async_copy(
                table_hbm.at[idx_v.at[h]], rows[p], gsem[p]).wait()

        def start_store(h, p):
            pltpu.make_async_copy(
                outt[p], out_hbm.at[h, :, wid], ssem[p]).start()

        def wait_store(h, p):
            pltpu.make_async_copy(
                outt[p], out_hbm.at[h, :, wid], ssem[p]).wait()

        def transpose(p):
            @pl.loop(0, 16, unroll=2)
            def _(k):
                m = jnp.bitwise_and(iota + k, 15)
                colv = [m + e16 * 16 for e16 in range(DIM // 16)]
                trv = [jnp.right_shift(m, 3) + e16 * 2
                       for e16 in range(DIM // 16)]
                cbase = jnp.left_shift(jnp.bitwise_and(m, 7), 7) + iota
                prev = None
                for b16 in range(BLK // 16):
                    cur = (cbase + b16 * 16,
                           [plsc.load_gather(rows[p], [rowv[b16], colv[e16]])
                            for e16 in range(DIM // 16)])
                    if prev is not None:
                        pcb, prvs = prev
                        for e16 in range(DIM // 16):
                            plsc.store_scatter(outt[p], [trv[e16], pcb],
                                               prvs[e16])
                    prev = cur
                pcb, prvs = prev
                for e16 in range(DIM // 16):
                    plsc.store_scatter(outt[p], [trv[e16], pcb], prvs[e16])

        start_gather(0, 0)
        start_gather(1, 1)

        @pl.loop(0, hist // 2)
        def _(g):
            for par in (0, 1):
                h = 2 * g + par

                wait_gather(h, par)

                @pl.when(g > 0)
                def _():
                    wait_store(h - 2, par)

                transpose(par)
                start_store(h, par)

                @pl.when(h + 2 < hist)
                def _():
                    start_gather(h + 2, par)

        wait_store(hist - 2, 0)
        wait_store(hist - 1, 1)

    return gather_kernel


@jax.jit
def kernel(x, action_emb_weight):
    b, h = x.shape
    xt = jnp.transpose(x, (1, 0)).astype(jnp.int32)
    out5 = _make_gather(b, h)(xt, action_emb_weight)
    out = out5.reshape(h, DIM // 8, NW, 8, BLK)
    return jnp.transpose(out, (2, 4, 0, 1, 3)).reshape(b, h, DIM)


# speedup vs baseline: 2.6649x; 1.0029x over previous
import functools

import jax
import jax.numpy as jnp
from jax import lax
from jax.experimental import pallas as pl
from jax.experimental.pallas import tpu as pltpu
from jax.experimental.pallas import tpu_sc as plsc

DIM = 64
NC = 2
NS = 16
NW = NC * NS
BLK = 128


def _make_gather(batch, hist):
    assert batch == NW * BLK and DIM == 64
    mesh = plsc.VectorSubcoreMesh(core_axis_name="c", subcore_axis_name="s")

    scratch = [
        pltpu.VMEM((hist, BLK), jnp.int32),
        pltpu.VMEM((BLK, DIM), jnp.float32),
        pltpu.VMEM((BLK, DIM), jnp.float32),
        pltpu.VMEM((DIM // 8, 8 * BLK), jnp.float32),
        pltpu.VMEM((DIM // 8, 8 * BLK), jnp.float32),
        pltpu.SemaphoreType.DMA,
        pltpu.SemaphoreType.DMA,
        pltpu.SemaphoreType.DMA,
        pltpu.SemaphoreType.DMA,
    ]

    @functools.partial(
        pl.kernel,
        out_type=jax.ShapeDtypeStruct((hist, DIM // 8, NW, 8 * BLK),
                                      jnp.float32),
        mesh=mesh,
        scratch_types=scratch,
        compiler_params=pltpu.CompilerParams(
            use_tc_tiling_on_sc=False, needs_layout_passes=False),
    )
    def gather_kernel(idx_hbm, table_hbm, out_hbm, idx_v,
                      rows0, rows1, outt0, outt1, g0, g1, s0, s1):
        rows = (rows0, rows1)
        outt = (outt0, outt1)
        gsem = (g0, g1)
        ssem = (s0, s1)
        wid = lax.axis_index("s") * NC + lax.axis_index("c")

        pltpu.sync_copy(idx_hbm.at[:, pl.ds(wid * BLK, BLK)], idx_v)

        iota = lax.broadcasted_iota(jnp.int32, (16,), 0)
        rowv = [iota + b16 * 16 for b16 in range(BLK // 16)]

        def start_gather(h, p):
            pltpu.make_async_copy(
                table_hbm.at[idx_v.at[h]], rows[p], gsem[p]).start()

        def wait_gather(h, p):
            pltpu.make_async_copy(
                table_hbm.at[idx_v.at[h]], rows[p], gsem[p]).wait()

        def start_store(h, p):
            pltpu.make_async_copy(
                outt[p], out_hbm.at[h, :, wid], ssem[p]).start()

        def wait_store(h, p):
            pltpu.make_async_copy(
                outt[p], out_hbm.at[h, :, wid], ssem[p]).wait()

        def transpose(p):
            @pl.loop(0, 16, unroll=4)
            def _(k):
                m = jnp.bitwise_and(iota + k, 15)
                colv = [m + e16 * 16 for e16 in range(DIM // 16)]
                trv = [jnp.right_shift(m, 3) + e16 * 2
                       for e16 in range(DIM // 16)]
                cbase = jnp.left_shift(jnp.bitwise_and(m, 7), 7) + iota
                prev = None
                for b16 in range(BLK // 16):
                    cur = (cbase + b16 * 16,
                           [plsc.load_gather(rows[p], [rowv[b16], colv[e16]])
                            for e16 in range(DIM // 16)])
                    if prev is not None:
                        pcb, prvs = prev
                        for e16 in range(DIM // 16):
                            plsc.store_scatter(outt[p], [trv[e16], pcb],
                                               prvs[e16])
                    prev = cur
                pcb, prvs = prev
                for e16 in range(DIM // 16):
                    plsc.store_scatter(outt[p], [trv[e16], pcb], prvs[e16])

        start_gather(0, 0)
        start_gather(1, 1)

        @pl.loop(0, hist // 2)
        def _(g):
            for par in (0, 1):
                h = 2 * g + par

                wait_gather(h, par)

                @pl.when(g > 0)
                def _():
                    wait_store(h - 2, par)

                transpose(par)
                start_store(h, par)

                @pl.when(h + 2 < hist)
                def _():
                    start_gather(h + 2, par)

        wait_store(hist - 2, 0)
        wait_store(hist - 1, 1)

    return gather_kernel


@jax.jit
def kernel(x, action_emb_weight):
    b, h = x.shape
    xt = jnp.transpose(x, (1, 0)).astype(jnp.int32)
    out5 = _make_gather(b, h)(xt, action_emb_weight)
    out = out5.reshape(h, DIM // 8, NW, 8, BLK)
    return jnp.transpose(out, (2, 4, 0, 1, 3)).reshape(b, h, DIM)


# speedup vs baseline: 2.8247x; 1.0599x over previous
import functools

import jax
import jax.numpy as jnp
from jax import lax
from jax.experimental import pallas as pl
from jax.experimental.pallas import tpu as pltpu
from jax.experimental.pallas import tpu_sc as plsc

DIM = 64
NC = 2
NS = 16
NW = NC * NS
BLK = 128


def _make_gather(batch, hist):
    assert batch == NW * BLK and DIM == 64
    mesh = plsc.VectorSubcoreMesh(core_axis_name="c", subcore_axis_name="s")

    nbuf = 4
    scratch = (
        [pltpu.VMEM((hist, BLK), jnp.int32)]
        + [pltpu.VMEM((BLK, DIM), jnp.float32) for _ in range(nbuf)]
        + [pltpu.VMEM((DIM // 8, 8 * BLK), jnp.float32) for _ in range(nbuf)]
        + [pltpu.SemaphoreType.DMA for _ in range(2 * nbuf)]
    )

    @functools.partial(
        pl.kernel,
        out_type=jax.ShapeDtypeStruct((hist, DIM // 8, NW, 8 * BLK),
                                      jnp.float32),
        mesh=mesh,
        scratch_types=scratch,
        compiler_params=pltpu.CompilerParams(
            use_tc_tiling_on_sc=False, needs_layout_passes=False),
    )
    def gather_kernel(idx_hbm, table_hbm, out_hbm, idx_v, *bufs):
        rows = bufs[:nbuf]
        outt = bufs[nbuf:2 * nbuf]
        gsem = bufs[2 * nbuf:3 * nbuf]
        ssem = bufs[3 * nbuf:]
        wid = lax.axis_index("s") * NC + lax.axis_index("c")

        pltpu.sync_copy(idx_hbm.at[:, pl.ds(wid * BLK, BLK)], idx_v)

        iota = lax.broadcasted_iota(jnp.int32, (16,), 0)
        rowv = [iota + b16 * 16 for b16 in range(BLK // 16)]

        def start_gather(h, p):
            pltpu.make_async_copy(
                table_hbm.at[idx_v.at[h]], rows[p], gsem[p]).start()

        def wait_gather(h, p):
            pltpu.make_async_copy(
                table_hbm.at[idx_v.at[h]], rows[p], gsem[p]).wait()

        def start_store(h, p):
            pltpu.make_async_copy(
                outt[p], out_hbm.at[h, :, wid], ssem[p]).start()

        def wait_store(h, p):
            pltpu.make_async_copy(
                outt[p], out_hbm.at[h, :, wid], ssem[p]).wait()

        def transpose(p):
            @pl.loop(0, 16, unroll=4)
            def _(k):
                m = jnp.bitwise_and(iota + k, 15)
                colv = [m + e16 * 16 for e16 in range(DIM // 16)]
                trv = [jnp.right_shift(m, 3) + e16 * 2
                       for e16 in range(DIM // 16)]
                cbase = jnp.left_shift(jnp.bitwise_and(m, 7), 7) + iota
                prev = None
                for b16 in range(BLK // 16):
                    cur = (cbase + b16 * 16,
                           [plsc.load_gather(rows[p], [rowv[b16], colv[e16]])
                            for e16 in range(DIM // 16)])
                    if prev is not None:
                        pcb, prvs = prev
                        for e16 in range(DIM // 16):
                            plsc.store_scatter(outt[p], [trv[e16], pcb],
                                               prvs[e16])
                    prev = cur
                pcb, prvs = prev
                for e16 in range(DIM // 16):
                    plsc.store_scatter(outt[p], [trv[e16], pcb], prvs[e16])

        for p in range(nbuf):
            start_gather(p, p)

        @pl.loop(0, hist // nbuf)
        def _(g):
            for par in range(nbuf):
                h = g * nbuf + par

                wait_gather(h, par)

                @pl.when(g > 0)
                def _():
                    wait_store(h - nbuf, par)

                transpose(par)
                start_store(h, par)

                @pl.when(h + nbuf < hist)
                def _():
                    start_gather(h + nbuf, par)

        for p in range(nbuf):
            wait_store(hist - nbuf + p, p)

    return gather_kernel


@jax.jit
def kernel(x, action_emb_weight):
    b, h = x.shape
    xt = jnp.transpose(x, (1, 0)).astype(jnp.int32)
    out5 = _make_gather(b, h)(xt, action_emb_weight)
    out = out5.reshape(h, DIM // 8, NW, 8, BLK)
    return jnp.transpose(out, (2, 4, 0, 1, 3)).reshape(b, h, DIM)
